# Initial kernel scaffold; baseline (speedup 1.0000x reference)
import jax
import jax.numpy as jnp
from jax.experimental import pallas as pl


def kernel(x, edge_index, edge_attr, batch, global_features, p_nodeW, p_nodeB, l0_linW, l0_attsrc, l0_attdst, l0_attedge, l0_edgeW, l0_convB, l0_lnW, l0_lnB, l0_projW, l0_projB, l1_linW, l1_attsrc, l1_attdst, l1_attedge, l1_edgeW, l1_convB, l1_lnW, l1_lnB, l2_linW, l2_attsrc, l2_attdst, l2_attedge, l2_edgeW, l2_convB, l2_lnW, l2_lnB, fc1W, fc1b, fc2W, fc2b, fc3W, fc3b, fc4W, fc4b):
    raise NotImplementedError("write your pallas kernel here")


# speedup vs baseline: 1.0088x; 1.0088x over previous
import functools
import jax
import jax.numpy as jnp
from jax.experimental import pallas as pl

_N = 50000
_B = 256
_H = 4
_C = 64
_HC = 256


def _mm(x, w, b, relu=False, block_m=2048):
    M, K = x.shape
    Kn, Nn = w.shape
    bm = min(block_m, max(8, -(-M // 8) * 8))
    pad_m = (-M) % bm
    if pad_m:
        x = jnp.pad(x, ((0, pad_m), (0, 0)))
    Mp = M + pad_m

    def kern(x_ref, w_ref, b_ref, o_ref):
        acc = jnp.dot(x_ref[...], w_ref[...],
                      preferred_element_type=jnp.float32) + b_ref[...]
        if relu:
            acc = jnp.maximum(acc, 0.0)
        o_ref[...] = acc

    out = pl.pallas_call(
        kern,
        grid=(Mp // bm,),
        in_specs=[
            pl.BlockSpec((bm, K), lambda i: (i, 0)),
            pl.BlockSpec((Kn, Nn), lambda i: (0, 0)),
            pl.BlockSpec((1, Nn), lambda i: (0, 0)),
        ],
        out_specs=pl.BlockSpec((bm, Nn), lambda i: (i, 0)),
        out_shape=jax.ShapeDtypeStruct((Mp, Nn), jnp.float32),
    )(x, w, b.reshape(1, -1))
    return out[:M]


def _blockdiag_att(att):
    return (jnp.eye(_H, dtype=att.dtype)[:, None, :] * att[:, :, None]).reshape(_H * _C, _H)


def _edge_att_mat(edgeW, aedge):
    return (edgeW.reshape(edgeW.shape[0], _H, _C) * aedge[None]).sum(-1)


def _layer_norm(x, w, b):
    mu = jnp.mean(x, axis=-1, keepdims=True)
    var = jnp.mean((x - mu) ** 2, axis=-1, keepdims=True)
    return (x - mu) / jnp.sqrt(var + 1e-5) * w + b


def _gat_sparse(hp, asn, adn, ae, src, dst, bias):
    alpha = asn[src] + adn[dst] + ae
    alpha = jnp.where(alpha >= 0, alpha, 0.2 * alpha)
    amax = jax.ops.segment_max(alpha, dst, num_segments=_N)
    amax = jnp.where(jnp.isfinite(amax), amax, 0.0)
    ex = jnp.exp(alpha - amax[dst])
    den = jax.ops.segment_sum(ex, dst, num_segments=_N)
    a = ex / (den[dst] + 1e-16)
    msg = hp[src].reshape(-1, _H, _C) * a[:, :, None]
    out = jax.ops.segment_sum(msg, dst, num_segments=_N)
    return out.reshape(_N, _HC) + bias


def kernel(x, edge_index, edge_attr, batch, global_features, p_nodeW, p_nodeB,
           l0_linW, l0_attsrc, l0_attdst, l0_attedge, l0_edgeW, l0_convB, l0_lnW, l0_lnB,
           l0_projW, l0_projB,
           l1_linW, l1_attsrc, l1_attdst, l1_attedge, l1_edgeW, l1_convB, l1_lnW, l1_lnB,
           l2_linW, l2_attsrc, l2_attdst, l2_attedge, l2_edgeW, l2_convB, l2_lnW, l2_lnB,
           fc1W, fc1b, fc2W, fc2b, fc3W, fc3b, fc4W, fc4b):
    n = x.shape[0]
    loop = jnp.arange(n, dtype=edge_index.dtype)
    src = jnp.concatenate([edge_index[0], loop])
    dst = jnp.concatenate([edge_index[1], loop])
    mean_ea = jnp.mean(edge_attr, axis=0, keepdims=True)
    ea_full = jnp.concatenate(
        [edge_attr, jnp.broadcast_to(mean_ea, (n, edge_attr.shape[1]))], axis=0)

    layers = [
        (l0_linW, l0_attsrc, l0_attdst, l0_attedge, l0_edgeW, l0_convB, l0_lnW, l0_lnB),
        (l1_linW, l1_attsrc, l1_attdst, l1_attedge, l1_edgeW, l1_convB, l1_lnW, l1_lnB),
        (l2_linW, l2_attsrc, l2_attdst, l2_attedge, l2_edgeW, l2_convB, l2_lnW, l2_lnB),
    ]

    Wae = jnp.concatenate([_edge_att_mat(lw[4], lw[3]) for lw in layers], axis=1)
    ae_all = _mm(ea_full, Wae, jnp.zeros((Wae.shape[1],), jnp.float32), block_m=8192)

    h = _mm(x, p_nodeW, p_nodeB)

    for i, (linW, asrc, adst, aedge, edgeW, convB, lnW, lnB) in enumerate(layers):
        cols = [linW, linW @ _blockdiag_att(asrc), linW @ _blockdiag_att(adst)]
        bias_parts = [jnp.zeros((_HC + 2 * _H,), jnp.float32)]
        if i == 0:
            cols.append(l0_projW)
            bias_parts.append(l0_projB)
        Wbig = jnp.concatenate(cols, axis=1)
        bbig = jnp.concatenate(bias_parts)
        out = _mm(h, Wbig, bbig)
        hp = out[:, :_HC]
        asn = out[:, _HC:_HC + _H]
        adn = out[:, _HC + _H:_HC + 2 * _H]
        res = out[:, _HC + 2 * _H:] if i == 0 else h
        ae = ae_all[:, i * _H:(i + 1) * _H]
        g = _gat_sparse(hp, asn, adn, ae, src, dst, convB)
        h = jnp.maximum(_layer_norm(g + res, lnW, lnB), 0.0)

    ones = jnp.ones((n,), jnp.float32)
    cnt = jax.ops.segment_sum(ones, batch, num_segments=_B)
    mean_pool = jax.ops.segment_sum(h, batch, num_segments=_B) / jnp.maximum(cnt, 1.0)[:, None]
    max_pool = jax.ops.segment_max(h, batch, num_segments=_B)
    rep = jnp.concatenate([mean_pool, max_pool, global_features], axis=-1)
    z = _mm(rep, fc1W, fc1b, relu=True)
    z = _mm(z, fc2W, fc2b, relu=True)
    z = _mm(z, fc3W, fc3b, relu=True)
    return _mm(z, fc4W, fc4b)


# speedup vs baseline: 12.8674x; 12.7554x over previous
import functools
import jax
import jax.numpy as jnp
from jax import lax
from jax.experimental import pallas as pl
from jax.experimental.pallas import tpu as pltpu
from jax.experimental.pallas import tpu_sc as plsc

_N = 50000
_B = 256
_H = 4
_C = 64
_HC = 256

_R = 256
_NCHUNK = -(-_N // _R)
_NPAD = _NCHUNK * _R
_ROW = 272
_G = 32
_NW = 32


def _mm(x, w, b, relu=False, block_m=2048):
    M, K = x.shape
    Kn, Nn = w.shape
    bm = min(block_m, max(8, -(-M // 8) * 8))
    pad_m = (-M) % bm
    if pad_m:
        x = jnp.pad(x, ((0, pad_m), (0, 0)))
    Mp = M + pad_m

    def kern(x_ref, w_ref, b_ref, o_ref):
        acc = jnp.dot(x_ref[...], w_ref[...], precision=lax.Precision.HIGHEST,
                      preferred_element_type=jnp.float32) + b_ref[...]
        if relu:
            acc = jnp.maximum(acc, 0.0)
        o_ref[...] = acc

    out = pl.pallas_call(
        kern,
        grid=(Mp // bm,),
        in_specs=[
            pl.BlockSpec((bm, K), lambda i: (i, 0)),
            pl.BlockSpec((Kn, Nn), lambda i: (0, 0)),
            pl.BlockSpec((1, Nn), lambda i: (0, 0)),
        ],
        out_specs=pl.BlockSpec((bm, Nn), lambda i: (i, 0)),
        out_shape=jax.ShapeDtypeStruct((Mp, Nn), jnp.float32),
    )(x, w, b.reshape(1, -1))
    return out[:M]


def _blockdiag_att(att):
    return (jnp.eye(_H, dtype=att.dtype)[:, None, :] * att[:, :, None]).reshape(_H * _C, _H)


def _edge_att_mat(edgeW, aedge):
    return (edgeW.reshape(edgeW.shape[0], _H, _C) * aedge[None]).sum(-1)


def _exp_f32(x):
    x = jnp.clip(x, -80.0, 80.0)
    magic = jnp.float32(12582912.0)
    kf = (x * jnp.float32(1.4426950408889634) + magic) - magic
    r = x - kf * jnp.float32(0.693359375)
    r = r - kf * jnp.float32(-2.12194440e-4)
    p = jnp.float32(1.0 / 120.0)
    p = p * r + jnp.float32(1.0 / 24.0)
    p = p * r + jnp.float32(1.0 / 6.0)
    p = p * r + jnp.float32(0.5)
    p = p * r + jnp.float32(1.0)
    p = p * r + jnp.float32(1.0)
    ki = kf.astype(jnp.int32)
    scale = jax.lax.bitcast_convert_type(
        jax.lax.shift_left(ki + 127, 23), jnp.float32)
    return p * scale

def _sc_gat_body(hpx, adn, srcs, dsts, aes, bounds, out,
                 src_v, dst_v, ae_v, rows_v, adn_v, num_v, bounds_v, sem):
    wid = lax.axis_index("s") * 2 + lax.axis_index("c")
    pltpu.sync_copy(bounds, bounds_v)
    lanes = lax.iota(jnp.int32, 16)
    nmy = (_NCHUNK - 1 - wid) // _NW + 1

    def chunk_body(ci, _):
        c = wid + ci * _NW
        base = c * _R

        def zero_body(i, _):
            for j in range(_ROW // 16):
                num_v[i, pl.ds(j * 16, 16)] = jnp.zeros((16,), jnp.float32)
            return 0
        lax.fori_loop(0, _R, zero_body, 0)

        pltpu.sync_copy(adn.at[pl.ds(base, _R)], adn_v)

        cidx = jnp.where(lanes == 1, c + 1, c).astype(jnp.int32)
        bv = plsc.load_gather(bounds_v, [cidx])
        b0 = bv[0]
        b1 = bv[1]
        b0r = (b0 // 8) * 8
        ntrip = (b1 - b0r + _G - 1) // _G

        def blk_body(k, _):
            e0 = b0r + k * _G
            pltpu.sync_copy(srcs.at[pl.ds(e0, _G)], src_v)
            pltpu.sync_copy(dsts.at[pl.ds(e0, _G)], dst_v)
            pltpu.sync_copy(aes.at[pl.ds(e0, _G)], ae_v)
            pltpu.async_copy(hpx.at[src_v], rows_v, sem).wait()

            for e in range(_G):
                sub, lane_e = divmod(e, 16)
                if lane_e == 0:
                    dvals = dst_v[pl.ds(sub * 16, 16)]
                dstl = dvals[lane_e] - base
                ok = (dstl >= 0) & (dstl < _R)
                dstl_c = jnp.clip(dstl, 0, _R - 1)
                adn_e = adn_v[dstl_c]
                alpha = rows_v[e, pl.ds(256, 16)] + adn_e + ae_v[e]
                alpha = jnp.where(alpha >= 0, alpha, 0.2 * alpha)
                p = jnp.where((lanes < 4) & ok, _exp_f32(alpha), 0.0)
                plsc.addupdate(num_v.at[dstl_c, pl.ds(256, 16)], p)
                for h in range(_H):
                    ph = p[h]
                    for cc in range(4):
                        col = h * 64 + cc * 16
                        val = rows_v[e, pl.ds(col, 16)] * ph
                        plsc.addupdate(num_v.at[dstl_c, pl.ds(col, 16)], val)
            return 0

        lax.fori_loop(0, ntrip, blk_body, 0)
        pltpu.sync_copy(num_v, out.at[pl.ds(base, _R)])
        return 0

    lax.fori_loop(0, nmy, chunk_body, 0)


_sc_gat = functools.partial(
    pl.kernel,
    mesh=plsc.VectorSubcoreMesh(core_axis_name="c", subcore_axis_name="s"),
    out_type=jax.ShapeDtypeStruct((_NPAD, _ROW), jnp.float32),
    compiler_params=pltpu.CompilerParams(needs_layout_passes=False,
                                         use_tc_tiling_on_sc=False),
    scratch_types=[
        pltpu.VMEM((_G,), jnp.int32),
        pltpu.VMEM((_G,), jnp.int32),
        pltpu.VMEM((_G, 16), jnp.float32),
        pltpu.VMEM((_G, _ROW), jnp.float32),
        pltpu.VMEM((_R, 16), jnp.float32),
        pltpu.VMEM((_R, _ROW), jnp.float32),
        pltpu.VMEM((256,), jnp.int32),
        pltpu.SemaphoreType.DMA,
    ],
)(_sc_gat_body)


def _finalize(num_flat, res, convB, lnW, lnB):
    num2d = num_flat
    bm = 1024

    def kern(num_ref, res_ref, cb_ref, lw_ref, lb_ref, o_ref):
        blk = num_ref[...]
        num = blk[:, :_HC]
        den = blk[:, _HC:_HC + _H]
        den_r = jnp.broadcast_to(den[:, :, None], (bm, _H, _C)).reshape(bm, _HC)
        g = num / (den_r + 1e-16) + cb_ref[...] + res_ref[...]
        mu = jnp.mean(g, axis=-1, keepdims=True)
        var = jnp.mean((g - mu) ** 2, axis=-1, keepdims=True)
        g = (g - mu) / jnp.sqrt(var + 1e-5) * lw_ref[...] + lb_ref[...]
        o_ref[...] = jnp.maximum(g, 0.0)

    return pl.pallas_call(
        kern,
        grid=(_NPAD // bm,),
        in_specs=[
            pl.BlockSpec((bm, _ROW), lambda i: (i, 0)),
            pl.BlockSpec((bm, _HC), lambda i: (i, 0)),
            pl.BlockSpec((1, _HC), lambda i: (0, 0)),
            pl.BlockSpec((1, _HC), lambda i: (0, 0)),
            pl.BlockSpec((1, _HC), lambda i: (0, 0)),
        ],
        out_specs=pl.BlockSpec((bm, _HC), lambda i: (i, 0)),
        out_shape=jax.ShapeDtypeStruct((_NPAD, _HC), jnp.float32),
    )(num2d, res, convB.reshape(1, -1), lnW.reshape(1, -1), lnB.reshape(1, -1))


def kernel(x, edge_index, edge_attr, batch, global_features, p_nodeW, p_nodeB,
           l0_linW, l0_attsrc, l0_attdst, l0_attedge, l0_edgeW, l0_convB, l0_lnW, l0_lnB,
           l0_projW, l0_projB,
           l1_linW, l1_attsrc, l1_attdst, l1_attedge, l1_edgeW, l1_convB, l1_lnW, l1_lnB,
           l2_linW, l2_attsrc, l2_attdst, l2_attedge, l2_edgeW, l2_convB, l2_lnW, l2_lnB,
           fc1W, fc1b, fc2W, fc2b, fc3W, fc3b, fc4W, fc4b):
    n = x.shape[0]
    loop = jnp.arange(n, dtype=edge_index.dtype)
    src = jnp.concatenate([edge_index[0], loop]).astype(jnp.int32)
    dst = jnp.concatenate([edge_index[1], loop]).astype(jnp.int32)
    mean_ea = jnp.mean(edge_attr, axis=0, keepdims=True)
    ea_full = jnp.concatenate(
        [edge_attr, jnp.broadcast_to(mean_ea, (n, edge_attr.shape[1]))], axis=0)

    layers = [
        (l0_linW, l0_attsrc, l0_attdst, l0_attedge, l0_edgeW, l0_convB, l0_lnW, l0_lnB),
        (l1_linW, l1_attsrc, l1_attdst, l1_attedge, l1_edgeW, l1_convB, l1_lnW, l1_lnB),
        (l2_linW, l2_attsrc, l2_attdst, l2_attedge, l2_edgeW, l2_convB, l2_lnW, l2_lnB),
    ]

    Wae = jnp.concatenate([_edge_att_mat(lw[4], lw[3]) for lw in layers], axis=1)
    ae_all = _mm(ea_full, Wae, jnp.zeros((Wae.shape[1],), jnp.float32), block_m=8192)

    ecnt = src.shape[0]
    ep = -(-ecnt // _G) * _G + 64
    perm = jnp.argsort(dst)
    src_s = jnp.pad(src[perm], (0, ep - ecnt))
    dst_s = jnp.pad(dst[perm], (0, ep - ecnt), constant_values=_NPAD)
    ae_s = jnp.pad(ae_all[perm], ((0, ep - ecnt), (0, 4)))
    bounds = jnp.searchsorted(dst_s, jnp.arange(_NCHUNK + 1, dtype=jnp.int32) * _R,
                              method='compare_all').astype(jnp.int32)
    bounds = jnp.pad(bounds, (0, 256 - _NCHUNK - 1))

    h = _mm(x, p_nodeW, p_nodeB)

    for i, lw in enumerate(layers):
        linW, asrc, adst, aedge, edgeW, convB, lnW, lnB = lw
        Whpx = jnp.concatenate(
            [linW, linW @ _blockdiag_att(asrc),
             jnp.zeros((linW.shape[0], 12), jnp.float32)], axis=1)
        Wadn = jnp.concatenate(
            [linW @ _blockdiag_att(adst),
             jnp.zeros((linW.shape[0], 12), jnp.float32)], axis=1)
        hpx = _mm(h, Whpx, jnp.zeros((_ROW,), jnp.float32))[:_N]
        adn = _mm(h, Wadn, jnp.zeros((16,), jnp.float32))
        adn = jnp.pad(adn[:_N], ((0, _NPAD - _N), (0, 0)))
        if i == 0:
            res = _mm(h, l0_projW, l0_projB)
        else:
            res = h
        res = jnp.pad(res[:_N], ((0, _NPAD - _N), (0, 0)))
        ae_l = jnp.concatenate(
            [ae_s[:, i * _H:(i + 1) * _H], jnp.zeros((ep, 12), jnp.float32)], axis=1)
        num_flat = _sc_gat(hpx, adn, src_s, dst_s, ae_l, bounds)
        h = _finalize(num_flat, res, convB, lnW, lnB)

    h = h[:_N]
    ones = jnp.ones((n,), jnp.float32)
    cnt = jax.ops.segment_sum(ones, batch, num_segments=_B)
    mean_pool = jax.ops.segment_sum(h, batch, num_segments=_B) / jnp.maximum(cnt, 1.0)[:, None]
    max_pool = jax.ops.segment_max(h, batch, num_segments=_B)
    rep = jnp.concatenate([mean_pool, max_pool, global_features], axis=-1)
    z = _mm(rep, fc1W, fc1b, relu=True)
    z = _mm(z, fc2W, fc2b, relu=True)
    z = _mm(z, fc3W, fc3b, relu=True)
    return _mm(z, fc4W, fc4b)


# speedup vs baseline: 15.1893x; 1.1804x over previous
import functools
import jax
import jax.numpy as jnp
from jax import lax
from jax.experimental import pallas as pl
from jax.experimental.pallas import tpu as pltpu
from jax.experimental.pallas import tpu_sc as plsc

_N = 50000
_B = 256
_H = 4
_C = 64
_HC = 256

_R = 256
_NCHUNK = -(-_N // _R)
_NPAD = _NCHUNK * _R
_ROW = 272
_G = 128
_NW = 32


def _mm(x, w, b, relu=False, block_m=2048):
    M, K = x.shape
    Kn, Nn = w.shape
    bm = min(block_m, max(8, -(-M // 8) * 8))
    pad_m = (-M) % bm
    if pad_m:
        x = jnp.pad(x, ((0, pad_m), (0, 0)))
    Mp = M + pad_m

    def kern(x_ref, w_ref, b_ref, o_ref):
        acc = jnp.dot(x_ref[...], w_ref[...], precision=lax.Precision.HIGHEST,
                      preferred_element_type=jnp.float32) + b_ref[...]
        if relu:
            acc = jnp.maximum(acc, 0.0)
        o_ref[...] = acc

    out = pl.pallas_call(
        kern,
        grid=(Mp // bm,),
        in_specs=[
            pl.BlockSpec((bm, K), lambda i: (i, 0)),
            pl.BlockSpec((Kn, Nn), lambda i: (0, 0)),
            pl.BlockSpec((1, Nn), lambda i: (0, 0)),
        ],
        out_specs=pl.BlockSpec((bm, Nn), lambda i: (i, 0)),
        out_shape=jax.ShapeDtypeStruct((Mp, Nn), jnp.float32),
    )(x, w, b.reshape(1, -1))
    return out[:M]


def _blockdiag_att(att):
    return (jnp.eye(_H, dtype=att.dtype)[:, None, :] * att[:, :, None]).reshape(_H * _C, _H)


def _edge_att_mat(edgeW, aedge):
    return (edgeW.reshape(edgeW.shape[0], _H, _C) * aedge[None]).sum(-1)


def _exp_f32(x):
    x = jnp.clip(x, -80.0, 80.0)
    magic = jnp.float32(12582912.0)
    kf = (x * jnp.float32(1.4426950408889634) + magic) - magic
    r = x - kf * jnp.float32(0.693359375)
    r = r - kf * jnp.float32(-2.12194440e-4)
    p = jnp.float32(1.0 / 120.0)
    p = p * r + jnp.float32(1.0 / 24.0)
    p = p * r + jnp.float32(1.0 / 6.0)
    p = p * r + jnp.float32(0.5)
    p = p * r + jnp.float32(1.0)
    p = p * r + jnp.float32(1.0)
    ki = kf.astype(jnp.int32)
    scale = jax.lax.bitcast_convert_type(
        jax.lax.shift_left(ki + 127, 23), jnp.float32)
    return p * scale

def _sc_gat_body(hpx, adn, srcs, dsts, aes, bounds, out,
                 src_v, dst_v, ae_v, rows_v, adn_v, num_v, bounds_v, sem):
    wid = lax.axis_index("s") * 2 + lax.axis_index("c")
    pltpu.sync_copy(bounds, bounds_v)
    lanes = lax.iota(jnp.int32, 16)
    nmy = (_NCHUNK - 1 - wid) // _NW + 1

    def chunk_body(ci, _):
        c = wid + ci * _NW
        base = c * _R

        def zero_body(i, _):
            for j in range(_ROW // 16):
                num_v[i, pl.ds(j * 16, 16)] = jnp.zeros((16,), jnp.float32)
            return 0
        lax.fori_loop(0, _R, zero_body, 0)

        pltpu.sync_copy(adn.at[pl.ds(base, _R)], adn_v)

        cidx = jnp.where(lanes == 1, c + 1, c).astype(jnp.int32)
        bv = plsc.load_gather(bounds_v, [cidx])
        b0 = bv[0]
        b1 = bv[1]
        b0r = (b0 // 8) * 8
        ntrip = (b1 - b0r + _G - 1) // _G

        def blk_body(k, _):
            e0 = b0r + k * _G
            pltpu.sync_copy(srcs.at[pl.ds(e0, _G)], src_v)
            pltpu.sync_copy(dsts.at[pl.ds(e0, _G)], dst_v)
            pltpu.sync_copy(aes.at[pl.ds(e0, _G)], ae_v)
            pltpu.async_copy(hpx.at[src_v], rows_v, sem).wait()

            def grp_body(g, _):
                gbase = g * 32
                for eu in range(32):
                    sub, lane_e = divmod(eu, 16)
                    if lane_e == 0:
                        dvals = dst_v[pl.ds(gbase + sub * 16, 16)]
                    e = gbase + eu
                    dstl = dvals[lane_e] - base
                    ok = (dstl >= 0) & (dstl < _R)
                    dstl_c = jnp.clip(dstl, 0, _R - 1)
                    adn_e = adn_v[dstl_c]
                    alpha = rows_v[e, pl.ds(256, 16)] + adn_e + ae_v[e]
                    alpha = jnp.where(alpha >= 0, alpha, 0.2 * alpha)
                    p = jnp.where((lanes < 4) & ok, _exp_f32(alpha), 0.0)
                    plsc.addupdate(num_v.at[dstl_c, pl.ds(256, 16)], p)
                    for h in range(_H):
                        ph = p[h]
                        for cc in range(4):
                            col = h * 64 + cc * 16
                            val = rows_v[e, pl.ds(col, 16)] * ph
                            plsc.addupdate(num_v.at[dstl_c, pl.ds(col, 16)], val)
                return 0

            lax.fori_loop(0, _G // 32, grp_body, 0)
            return 0

        lax.fori_loop(0, ntrip, blk_body, 0)
        pltpu.sync_copy(num_v, out.at[pl.ds(base, _R)])
        return 0

    lax.fori_loop(0, nmy, chunk_body, 0)


_sc_gat = functools.partial(
    pl.kernel,
    mesh=plsc.VectorSubcoreMesh(core_axis_name="c", subcore_axis_name="s"),
    out_type=jax.ShapeDtypeStruct((_NPAD, _ROW), jnp.float32),
    compiler_params=pltpu.CompilerParams(needs_layout_passes=False,
                                         use_tc_tiling_on_sc=False),
    scratch_types=[
        pltpu.VMEM((_G,), jnp.int32),
        pltpu.VMEM((_G,), jnp.int32),
        pltpu.VMEM((_G, 16), jnp.float32),
        pltpu.VMEM((_G, _ROW), jnp.float32),
        pltpu.VMEM((_R, 16), jnp.float32),
        pltpu.VMEM((_R, _ROW), jnp.float32),
        pltpu.VMEM((256,), jnp.int32),
        pltpu.SemaphoreType.DMA,
    ],
)(_sc_gat_body)


def _sc_pool_body(h_hbm, gbounds, out, rows_v, stage_v, gb_v, sem):
    wid = lax.axis_index("s") * 2 + lax.axis_index("c")
    pltpu.sync_copy(gbounds, gb_v)
    lanes = lax.iota(jnp.int32, 16)

    def graph_body(gi, _):
        b = wid + gi * _NW
        cidx = jnp.where(lanes == 1, b + 1, b).astype(jnp.int32)
        bv = plsc.load_gather(gb_v, [cidx])
        gb0 = bv[0]
        gb1 = bv[1]
        gb0r = (gb0 // 8) * 8
        ntrip = (gb1 - gb0r + 15) // 16
        init = (tuple(jnp.zeros((16,), jnp.float32) for _ in range(16)),
                tuple(jnp.full((16,), -jnp.inf, jnp.float32) for _ in range(16)))

        def blk(k, carry):
            r0 = gb0r + k * 16
            pltpu.sync_copy(h_hbm.at[pl.ds(r0, 16)], rows_v)
            sums = list(carry[0])
            maxs = list(carry[1])
            for r in range(16):
                valid = (r0 + r >= gb0) & (r0 + r < gb1)
                for j in range(16):
                    row = rows_v[r, pl.ds(j * 16, 16)]
                    sums[j] = sums[j] + jnp.where(valid, row, 0.0)
                    maxs[j] = jnp.where(valid, jnp.maximum(maxs[j], row), maxs[j])
            return (tuple(sums), tuple(maxs))

        sums, maxs = lax.fori_loop(0, ntrip, blk, init)
        denom = jnp.maximum((gb1 - gb0).astype(jnp.float32), 1.0)
        for j in range(16):
            stage_v[pl.ds(j * 16, 16)] = sums[j] / denom
            stage_v[pl.ds(_HC + j * 16, 16)] = maxs[j]
        pltpu.sync_copy(stage_v, out.at[b])
        return 0

    lax.fori_loop(0, _B // _NW, graph_body, 0)


_sc_pool = functools.partial(
    pl.kernel,
    mesh=plsc.VectorSubcoreMesh(core_axis_name="c", subcore_axis_name="s"),
    out_type=jax.ShapeDtypeStruct((_B, 2 * _HC), jnp.float32),
    compiler_params=pltpu.CompilerParams(needs_layout_passes=False,
                                         use_tc_tiling_on_sc=False),
    scratch_types=[
        pltpu.VMEM((16, _HC), jnp.float32),
        pltpu.VMEM((2 * _HC,), jnp.float32),
        pltpu.VMEM((272,), jnp.int32),
        pltpu.SemaphoreType.DMA,
    ],
)(_sc_pool_body)


def _finalize(num_flat, res, convB, lnW, lnB):
    num2d = num_flat
    bm = 1024

    def kern(num_ref, res_ref, cb_ref, lw_ref, lb_ref, o_ref):
        blk = num_ref[...]
        num = blk[:, :_HC]
        den = blk[:, _HC:_HC + _H]
        den_r = jnp.broadcast_to(den[:, :, None], (bm, _H, _C)).reshape(bm, _HC)
        g = num / (den_r + 1e-16) + cb_ref[...] + res_ref[...]
        mu = jnp.mean(g, axis=-1, keepdims=True)
        var = jnp.mean((g - mu) ** 2, axis=-1, keepdims=True)
        g = (g - mu) / jnp.sqrt(var + 1e-5) * lw_ref[...] + lb_ref[...]
        o_ref[...] = jnp.maximum(g, 0.0)

    return pl.pallas_call(
        kern,
        grid=(_NPAD // bm,),
        in_specs=[
            pl.BlockSpec((bm, _ROW), lambda i: (i, 0)),
            pl.BlockSpec((bm, _HC), lambda i: (i, 0)),
            pl.BlockSpec((1, _HC), lambda i: (0, 0)),
            pl.BlockSpec((1, _HC), lambda i: (0, 0)),
            pl.BlockSpec((1, _HC), lambda i: (0, 0)),
        ],
        out_specs=pl.BlockSpec((bm, _HC), lambda i: (i, 0)),
        out_shape=jax.ShapeDtypeStruct((_NPAD, _HC), jnp.float32),
    )(num2d, res, convB.reshape(1, -1), lnW.reshape(1, -1), lnB.reshape(1, -1))


def kernel(x, edge_index, edge_attr, batch, global_features, p_nodeW, p_nodeB,
           l0_linW, l0_attsrc, l0_attdst, l0_attedge, l0_edgeW, l0_convB, l0_lnW, l0_lnB,
           l0_projW, l0_projB,
           l1_linW, l1_attsrc, l1_attdst, l1_attedge, l1_edgeW, l1_convB, l1_lnW, l1_lnB,
           l2_linW, l2_attsrc, l2_attdst, l2_attedge, l2_edgeW, l2_convB, l2_lnW, l2_lnB,
           fc1W, fc1b, fc2W, fc2b, fc3W, fc3b, fc4W, fc4b):
    n = x.shape[0]
    loop = jnp.arange(n, dtype=edge_index.dtype)
    src = jnp.concatenate([edge_index[0], loop]).astype(jnp.int32)
    dst = jnp.concatenate([edge_index[1], loop]).astype(jnp.int32)
    mean_ea = jnp.mean(edge_attr, axis=0, keepdims=True)
    ea_full = jnp.concatenate(
        [edge_attr, jnp.broadcast_to(mean_ea, (n, edge_attr.shape[1]))], axis=0)

    layers = [
        (l0_linW, l0_attsrc, l0_attdst, l0_attedge, l0_edgeW, l0_convB, l0_lnW, l0_lnB),
        (l1_linW, l1_attsrc, l1_attdst, l1_attedge, l1_edgeW, l1_convB, l1_lnW, l1_lnB),
        (l2_linW, l2_attsrc, l2_attdst, l2_attedge, l2_edgeW, l2_convB, l2_lnW, l2_lnB),
    ]

    Wae = jnp.concatenate([_edge_att_mat(lw[4], lw[3]) for lw in layers], axis=1)
    ae_all = _mm(ea_full, Wae, jnp.zeros((Wae.shape[1],), jnp.float32), block_m=8192)

    ecnt = src.shape[0]
    ep = -(-ecnt // _G) * _G + 64
    perm = jnp.argsort(dst)
    src_s = jnp.pad(src[perm], (0, ep - ecnt))
    dst_s = jnp.pad(dst[perm], (0, ep - ecnt), constant_values=_NPAD)
    ae_s = jnp.pad(ae_all[perm], ((0, ep - ecnt), (0, 4)))
    bounds = jnp.searchsorted(dst_s, jnp.arange(_NCHUNK + 1, dtype=jnp.int32) * _R,
                              method='compare_all').astype(jnp.int32)
    bounds = jnp.pad(bounds, (0, 256 - _NCHUNK - 1))

    h = _mm(x, p_nodeW, p_nodeB)

    for i, lw in enumerate(layers):
        linW, asrc, adst, aedge, edgeW, convB, lnW, lnB = lw
        Whpx = jnp.concatenate(
            [linW, linW @ _blockdiag_att(asrc),
             jnp.zeros((linW.shape[0], 12), jnp.float32)], axis=1)
        Wadn = jnp.concatenate(
            [linW @ _blockdiag_att(adst),
             jnp.zeros((linW.shape[0], 12), jnp.float32)], axis=1)
        hpx = _mm(h, Whpx, jnp.zeros((_ROW,), jnp.float32))[:_N]
        adn = _mm(h, Wadn, jnp.zeros((16,), jnp.float32))
        adn = jnp.pad(adn[:_N], ((0, _NPAD - _N), (0, 0)))
        if i == 0:
            res = _mm(h, l0_projW, l0_projB)
        else:
            res = h
        res = jnp.pad(res[:_N], ((0, _NPAD - _N), (0, 0)))
        ae_l = jnp.concatenate(
            [ae_s[:, i * _H:(i + 1) * _H], jnp.zeros((ep, 12), jnp.float32)], axis=1)
        num_flat = _sc_gat(hpx, adn, src_s, dst_s, ae_l, bounds)
        h = _finalize(num_flat, res, convB, lnW, lnB)

    gb = jnp.searchsorted(batch.astype(jnp.int32),
                          jnp.arange(_B + 1, dtype=jnp.int32),
                          method='compare_all').astype(jnp.int32)
    pooled = _sc_pool(h, jnp.pad(gb, (0, 272 - _B - 1)))
    rep = jnp.concatenate([pooled, global_features], axis=-1)
    z = _mm(rep, fc1W, fc1b, relu=True)
    z = _mm(z, fc2W, fc2b, relu=True)
    z = _mm(z, fc3W, fc3b, relu=True)
    return _mm(z, fc4W, fc4b)
